# speedup vs baseline: 1.8073x; 1.8073x over previous
import functools

import jax
import jax.numpy as jnp
from jax import lax
from jax.experimental import pallas as pl
from jax.experimental.pallas import tpu as pltpu
from jax.experimental.pallas import tpu_sc as plsc

B = 16384
D = 128
LANES = 16
NC = 2
NS = 16
NW = NC * NS
BPW = B // NW
CH = 128
NCH = BPW // CH


def _make_sc_kernel():
    mesh = plsc.VectorSubcoreMesh(core_axis_name="c", subcore_axis_name="s")

    @functools.partial(
        pl.kernel,
        out_type=jax.ShapeDtypeStruct((B, D), jnp.float32),
        mesh=mesh,
        scratch_types=[
            pltpu.VMEM((NCH, CH), jnp.int32),
            pltpu.VMEM((CH, D), jnp.float32),
            pltpu.VMEM((CH, D), jnp.float32),
            pltpu.SemaphoreType.DMA,
            pltpu.SemaphoreType.DMA,
        ],
    )
    def k(z_hbm, label_hbm, table_hbm, out_hbm, idx_v, rows_v, z_v, gsem, zsem):
        wid = lax.axis_index("s") * NC + lax.axis_index("c")
        base = wid * BPW
        pltpu.sync_copy(label_hbm.at[wid], idx_v)
        for j in range(NCH):
            g = pltpu.async_copy(table_hbm.at[idx_v.at[j]], rows_v, gsem)
            zc = pltpu.async_copy(z_hbm.at[pl.ds(base + j * CH, CH)], z_v, zsem)
            g.wait()
            zc.wait()

            def mul_row(r, c):
                for kk in range(D // LANES):
                    s = pl.ds(kk * LANES, LANES)
                    rows_v[r, s] = rows_v[r, s] * z_v[r, s]
                return c

            lax.fori_loop(0, CH, mul_row, 0)
            pltpu.sync_copy(rows_v, out_hbm.at[pl.ds(base + j * CH, CH)])

    return k


_sc_kernel = _make_sc_kernel()


def kernel(z, label, emb_table):
    lab = label.astype(jnp.int32).reshape(NW, NCH, CH)
    return _sc_kernel(z, lab, emb_table)


# speedup vs baseline: 1.9558x; 1.0822x over previous
import functools

import jax
import jax.numpy as jnp
from jax import lax
from jax.experimental import pallas as pl
from jax.experimental.pallas import tpu as pltpu
from jax.experimental.pallas import tpu_sc as plsc

B = 16384
D = 128
LANES = 16
NC = 2
NS = 16
NW = NC * NS
BPW = B // NW
CH = 128
NCH = BPW // CH


def _make_sc_kernel():
    mesh = plsc.VectorSubcoreMesh(core_axis_name="c", subcore_axis_name="s")

    @functools.partial(
        pl.kernel,
        out_type=jax.ShapeDtypeStruct((B, D), jnp.float32),
        mesh=mesh,
        scratch_types=[
            pltpu.VMEM((NCH, CH), jnp.int32),
            pltpu.VMEM((CH, D), jnp.float32),
            pltpu.VMEM((CH, D), jnp.float32),
            pltpu.VMEM((CH, D), jnp.float32),
            pltpu.VMEM((CH, D), jnp.float32),
            pltpu.SemaphoreType.DMA,
            pltpu.SemaphoreType.DMA,
            pltpu.SemaphoreType.DMA,
            pltpu.SemaphoreType.DMA,
            pltpu.SemaphoreType.DMA,
            pltpu.SemaphoreType.DMA,
        ],
    )
    def k(z_hbm, label_hbm, table_hbm, out_hbm,
          idx_v, rows0, rows1, zv0, zv1,
          gsem0, gsem1, zsem0, zsem1, osem0, osem1):
        wid = lax.axis_index("s") * NC + lax.axis_index("c")
        base = wid * BPW
        rows = (rows0, rows1)
        zv = (zv0, zv1)
        gsem = (gsem0, gsem1)
        zsem = (zsem0, zsem1)
        osem = (osem0, osem1)

        pltpu.sync_copy(label_hbm.at[wid], idx_v)

        def start_in(j):
            b = j % 2
            g = pltpu.async_copy(table_hbm.at[idx_v.at[j]], rows[b], gsem[b])
            zc = pltpu.async_copy(z_hbm.at[pl.ds(base + j * CH, CH)], zv[b], zsem[b])
            return g, zc

        in_flight = start_in(0)
        stores = [None, None]
        for j in range(NCH):
            b = j % 2
            g, zc = in_flight
            g.wait()
            zc.wait()
            if j + 1 < NCH:
                nb = b ^ 1
                if stores[nb] is not None:
                    stores[nb].wait()
                    stores[nb] = None
                in_flight = start_in(j + 1)

            rows_b = rows[b]
            zv_b = zv[b]

            @plsc.parallel_loop(0, CH, unroll=2)
            def mul_row(r):
                for kk in range(D // LANES):
                    s = pl.ds(kk * LANES, LANES)
                    rows_b[r, s] = rows_b[r, s] * zv_b[r, s]

            stores[b] = pltpu.async_copy(
                rows_b, out_hbm.at[pl.ds(base + j * CH, CH)], osem[b])
        for st in stores:
            if st is not None:
                st.wait()

    return k


_sc_kernel = _make_sc_kernel()


def kernel(z, label, emb_table):
    lab = label.astype(jnp.int32).reshape(NW, NCH, CH)
    return _sc_kernel(z, lab, emb_table)
